# baseline (speedup 1.0000x reference)
import functools

import jax
import jax.numpy as jnp
from jax import lax
from jax.experimental import pallas as pl
from jax.experimental.pallas import tpu as pltpu
from jax.experimental.pallas import tpu_sc as plsc

NS = 10000
E = 640000
D = 128
H = 64
O = 2

NC = 2
NSUB = 16
NW = NC * NSUB
K = 128
NB = 8
G = 20
CH = NB * G
E_PAD = NW * K * CH
ROWS_W = E_PAD // K
N_PAD = 10016
CW = 16


def _seg_sum_kernel(with_count):
    mesh = plsc.VectorSubcoreMesh(core_axis_name="c", subcore_axis_name="s")
    out_type = [jax.ShapeDtypeStruct((NC, N_PAD, H), jnp.float32)]
    scratch = [
        pltpu.VMEM((NB, K), jnp.int32),
        pltpu.VMEM((NB, K), jnp.int32),
        pltpu.VMEM((NB, K), jnp.int32),
        pltpu.VMEM((NB, K), jnp.int32),
        pltpu.VMEM((NB * K, H), jnp.float32),
        pltpu.VMEM_SHARED((N_PAD, H), jnp.float32),
        pltpu.SemaphoreType.DMA,
        pltpu.SemaphoreType.DMA,
        pltpu.SemaphoreType.DMA,
        pltpu.SemaphoreType.DMA,
    ]
    if with_count:
        out_type.append(jax.ShapeDtypeStruct((NC, N_PAD, CW), jnp.float32))
        scratch += [
            pltpu.VMEM((K, CW), jnp.float32),
            pltpu.VMEM_SHARED((N_PAD, CW), jnp.float32),
        ]

    def body(*refs):
        if with_count:
            (y, srcs, dsts, z64, z16, ones_in, acc_out, cnt_out,
             sidxA, sidxB, didxA, didxB, rows, acc,
             semIA, semIB, semG, semS, ones_v, cnt) = refs
        else:
            (y, srcs, dsts, z64, acc_out,
             sidxA, sidxB, didxA, didxB, rows, acc,
             semIA, semIB, semG, semS) = refs
        cid = lax.axis_index("c")
        sid = lax.axis_index("s")
        wid = cid * NSUB + sid

        @pl.when(sid == 0)
        def _init():
            pltpu.sync_copy(z64, acc)
            if with_count:
                pltpu.sync_copy(z16, cnt)

        if with_count:
            pltpu.sync_copy(ones_in, ones_v)
        plsc.subcore_barrier()

        row0 = wid * (G * NB)

        def load_idx(grow, sidx, didx, sem):
            pltpu.async_copy(srcs.at[pl.ds(grow, NB)], sidx, sem)
            pltpu.async_copy(dsts.at[pl.ds(grow, NB)], didx, sem)

        def drain_idx(sidx, didx, sem):
            pltpu.make_async_copy(srcs.at[pl.ds(0, NB)], sidx, sem).wait()
            pltpu.make_async_copy(dsts.at[pl.ds(0, NB)], didx, sem).wait()

        def phase(sidx, didx, semI, prefetch_grow, prefetch_pred,
                  psidx, pdidx, psemI):
            drain_idx(sidx, didx, semI)
            gd = [
                pltpu.async_copy(
                    y.at[sidx.at[j]], rows.at[pl.ds(j * K, K)], semG)
                for j in range(NB)
            ]
            sd = []
            if with_count:
                sd += [
                    pltpu.async_copy(
                        ones_v, cnt.at[didx.at[j]], semS, add=True)
                    for j in range(NB)
                ]

            @pl.when(prefetch_pred)
            def _():
                load_idx(prefetch_grow, psidx, pdidx, psemI)

            for d in gd:
                d.wait()
            sd += [
                pltpu.async_copy(
                    rows.at[pl.ds(j * K, K)], acc.at[didx.at[j]],
                    semS, add=True)
                for j in range(NB)
            ]
            for d in sd:
                d.wait()

        load_idx(row0, sidxA, didxA, semIA)

        def group_pair(i, carry):
            g0row = row0 + (2 * i) * NB
            phase(sidxA, didxA, semIA,
                  g0row + NB, i >= 0, sidxB, didxB, semIB)
            phase(sidxB, didxB, semIB,
                  g0row + 2 * NB, i < (G // 2 - 1), sidxA, didxA, semIA)
            return carry

        lax.fori_loop(0, G // 2, group_pair, 0)
        plsc.subcore_barrier()

        @pl.when(sid == 0)
        def _export():
            pltpu.sync_copy(acc, acc_out.at[cid])
            if with_count:
                pltpu.sync_copy(cnt, cnt_out.at[cid])

    return pl.kernel(
        body,
        out_type=tuple(out_type) if with_count else out_type[0],
        mesh=mesh,
        scratch_types=scratch,
        compiler_params=pltpu.CompilerParams(use_tc_tiling_on_sc=False),
    )


def _pre_body(x_ref, wl_ref, wr_ref, y_ref, z_ref):
    x = x_ref[:]
    y_ref[:] = jnp.dot(x, wl_ref[:], preferred_element_type=jnp.float32)
    z_ref[:] = jnp.dot(x, wr_ref[:], preferred_element_type=jnp.float32)


def _mid_body(sp_ref, cp_ref, z_ref, bl_ref, wl_ref, wr_ref, y2_ref, z2_ref):
    s = sp_ref[0] + sp_ref[1]
    cnt = cp_ref[0, :, 0:1] + cp_ref[1, :, 0:1]
    mean = s / jnp.maximum(cnt, 1.0)
    h = jnp.maximum(mean + bl_ref[:] + z_ref[:], 0.0)
    y2_ref[:] = jnp.dot(h, wl_ref[:], preferred_element_type=jnp.float32)
    z2_ref[:] = jnp.dot(h, wr_ref[:], preferred_element_type=jnp.float32)


def _fin_body(sp_ref, cp_ref, z_ref, bl_ref, wlin_ref, blin_ref, out_ref):
    s = sp_ref[0] + sp_ref[1]
    cnt = cp_ref[0, :, 0:1] + cp_ref[1, :, 0:1]
    mean = s / jnp.maximum(cnt, 1.0)
    h = jnp.maximum(mean + bl_ref[:] + z_ref[:], 0.0)
    out_ref[:] = (
        jnp.dot(h, wlin_ref[:], preferred_element_type=jnp.float32)
        + blin_ref[:]
    )


_f32 = jnp.float32


def kernel(x_subject, x_roi, ei_s2r, ei_r2r, ei_s2s,
           c1_s2r_Wl, c1_s2r_bl, c1_s2r_Wr, c1_r2r_Wl, c1_r2r_bl, c1_r2r_Wr,
           c1_s2s_Wl, c1_s2s_bl, c1_s2s_Wr,
           c2_s2r_Wl, c2_s2r_bl, c2_s2r_Wr, c2_r2r_Wl, c2_r2r_bl, c2_r2r_Wr,
           c2_s2s_Wl, c2_s2s_bl, c2_s2s_Wr,
           lin_W, lin_b):
    xp = jnp.pad(x_subject, ((0, N_PAD - NS), (0, 0)))
    npad = E_PAD - E
    srcs = jnp.concatenate(
        [ei_s2s[0], jnp.full((npad,), NS, jnp.int32)]).reshape(ROWS_W, K)
    dsts = jnp.concatenate(
        [ei_s2s[1], jnp.full((npad,), N_PAD - 1, jnp.int32)]).reshape(ROWS_W, K)
    z64 = jnp.zeros((N_PAD, H), _f32)
    z16 = jnp.zeros((N_PAD, CW), _f32)
    ones = jnp.ones((K, CW), _f32)
    bl1 = c1_s2s_bl.reshape(1, H)
    bl2 = c2_s2s_bl.reshape(1, H)
    wlin = jnp.pad(lin_W, ((0, 0), (0, 128 - O)))
    blin = jnp.pad(lin_b, ((0, 128 - O))).reshape(1, 128)

    y1, z1 = pl.pallas_call(
        _pre_body,
        out_shape=[jax.ShapeDtypeStruct((N_PAD, H), _f32)] * 2,
    )(xp, c1_s2s_Wl, c1_s2s_Wr)

    s1p, cntp = _seg_sum_kernel(True)(y1, srcs, dsts, z64, z16, ones)

    y2, z2 = pl.pallas_call(
        _mid_body,
        out_shape=[jax.ShapeDtypeStruct((N_PAD, H), _f32)] * 2,
    )(s1p, cntp, z1, bl1, c2_s2s_Wl, c2_s2s_Wr)

    s2p = _seg_sum_kernel(False)(y2, srcs, dsts, z64)

    outp = pl.pallas_call(
        _fin_body,
        out_shape=jax.ShapeDtypeStruct((N_PAD, 128), _f32),
    )(s2p, cntp, z2, bl2, wlin, blin)

    return outp[:NS, :O]


# speedup vs baseline: 1.0009x; 1.0009x over previous
import functools

import jax
import jax.numpy as jnp
from jax import lax
from jax.experimental import pallas as pl
from jax.experimental.pallas import tpu as pltpu
from jax.experimental.pallas import tpu_sc as plsc

NS = 10000
E = 640000
D = 128
H = 64
O = 2

NC = 2
NSUB = 16
NW = NC * NSUB
K = 128
NB = 8
G = 20
CH = NB * G
E_PAD = NW * K * CH
ROWS_W = E_PAD // K
N_PAD = 10016
CW = 16


def _seg_sum_kernel(with_count):
    mesh = plsc.VectorSubcoreMesh(core_axis_name="c", subcore_axis_name="s")
    out_type = [jax.ShapeDtypeStruct((NC, N_PAD, H), jnp.float32)]
    scratch = [
        pltpu.VMEM((NB, K), jnp.int32),
        pltpu.VMEM((NB, K), jnp.int32),
        pltpu.VMEM((NB, K), jnp.int32),
        pltpu.VMEM((NB, K), jnp.int32),
        pltpu.VMEM((NB * K, H), jnp.float32),
        pltpu.VMEM_SHARED((N_PAD, H), jnp.float32),
        pltpu.SemaphoreType.DMA,
        pltpu.SemaphoreType.DMA,
        pltpu.SemaphoreType.DMA,
        pltpu.SemaphoreType.DMA,
    ]
    if with_count:
        out_type.append(jax.ShapeDtypeStruct((NC, N_PAD, CW), jnp.float32))
        scratch += [
            pltpu.VMEM((K, CW), jnp.float32),
            pltpu.VMEM_SHARED((N_PAD, CW), jnp.float32),
        ]

    def body(*refs):
        if with_count:
            (y, srcs, dsts, z64, z16, ones_in, acc_out, cnt_out,
             sidxA, sidxB, didxA, didxB, rows, acc,
             semIA, semIB, semG, semS, ones_v, cnt) = refs
        else:
            (y, srcs, dsts, z64, acc_out,
             sidxA, sidxB, didxA, didxB, rows, acc,
             semIA, semIB, semG, semS) = refs
        cid = lax.axis_index("c")
        sid = lax.axis_index("s")
        wid = cid * NSUB + sid

        @pl.when(sid == 0)
        def _init():
            pltpu.sync_copy(z64, acc)
            if with_count:
                pltpu.sync_copy(z16, cnt)

        if with_count:
            pltpu.sync_copy(ones_in, ones_v)
        plsc.subcore_barrier()

        row0 = wid * (G * NB)

        def load_idx(grow, sidx, didx, sem):
            pltpu.async_copy(srcs.at[pl.ds(grow, NB)], sidx, sem)
            pltpu.async_copy(dsts.at[pl.ds(grow, NB)], didx, sem)

        def drain_idx(sidx, didx, sem):
            pltpu.make_async_copy(srcs.at[pl.ds(0, NB)], sidx, sem).wait()
            pltpu.make_async_copy(dsts.at[pl.ds(0, NB)], didx, sem).wait()

        def phase(sidx, didx, semI, prefetch_grow, prefetch_pred,
                  psidx, pdidx, psemI):
            drain_idx(sidx, didx, semI)
            gd = [
                pltpu.async_copy(
                    y.at[sidx.at[j]], rows.at[pl.ds(j * K, K)], semG)
                for j in range(NB)
            ]
            sd = []
            if with_count:
                sd += [
                    pltpu.async_copy(
                        ones_v, cnt.at[didx.at[j]], semS, add=True)
                    for j in range(NB)
                ]

            @pl.when(prefetch_pred)
            def _():
                load_idx(prefetch_grow, psidx, pdidx, psemI)

            for d in gd:
                d.wait()
            sd += [
                pltpu.async_copy(
                    rows.at[pl.ds(j * K, K)], acc.at[didx.at[j]],
                    semS, add=True)
                for j in range(NB)
            ]
            for d in sd:
                d.wait()

        load_idx(row0, sidxA, didxA, semIA)

        def group_pair(i, carry):
            g0row = row0 + (2 * i) * NB
            phase(sidxA, didxA, semIA,
                  g0row + NB, i >= 0, sidxB, didxB, semIB)
            phase(sidxB, didxB, semIB,
                  g0row + 2 * NB, i < (G // 2 - 1), sidxA, didxA, semIA)
            return carry

        lax.fori_loop(0, G // 2, group_pair, 0)
        plsc.subcore_barrier()

        @pl.when(sid == 0)
        def _export():
            pltpu.sync_copy(acc, acc_out.at[cid])
            if with_count:
                pltpu.sync_copy(cnt, cnt_out.at[cid])

    return pl.kernel(
        body,
        out_type=tuple(out_type) if with_count else out_type[0],
        mesh=mesh,
        scratch_types=scratch,
        compiler_params=pltpu.CompilerParams(use_tc_tiling_on_sc=False),
    )


def _pre_body(x_ref, wl_ref, wr_ref, y_ref, z_ref):
    x = x_ref[:]
    y_ref[:] = jnp.dot(x, wl_ref[:], preferred_element_type=jnp.float32)
    z_ref[:] = jnp.dot(x, wr_ref[:], preferred_element_type=jnp.float32)


def _mid_body(sp_ref, cp_ref, z_ref, bl_ref, wl_ref, wr_ref, y2_ref, z2_ref):
    s = sp_ref[0] + sp_ref[1]
    cnt = cp_ref[0, :, 0:1] + cp_ref[1, :, 0:1]
    mean = s / jnp.maximum(cnt, 1.0)
    h = jnp.maximum(mean + bl_ref[:] + z_ref[:], 0.0)
    y2_ref[:] = jnp.dot(h, wl_ref[:], preferred_element_type=jnp.float32)
    z2_ref[:] = jnp.dot(h, wr_ref[:], preferred_element_type=jnp.float32)


def _fin_body(sp_ref, cp_ref, z_ref, bl_ref, wlin_ref, blin_ref, out_ref):
    s = sp_ref[0] + sp_ref[1]
    cnt = cp_ref[0, :, 0:1] + cp_ref[1, :, 0:1]
    mean = s / jnp.maximum(cnt, 1.0)
    h = jnp.maximum(mean + bl_ref[:] + z_ref[:], 0.0)
    out_ref[:] = (
        jnp.dot(h, wlin_ref[:], preferred_element_type=jnp.float32)
        + blin_ref[:]
    )


_f32 = jnp.float32


def kernel(x_subject, x_roi, ei_s2r, ei_r2r, ei_s2s,
           c1_s2r_Wl, c1_s2r_bl, c1_s2r_Wr, c1_r2r_Wl, c1_r2r_bl, c1_r2r_Wr,
           c1_s2s_Wl, c1_s2s_bl, c1_s2s_Wr,
           c2_s2r_Wl, c2_s2r_bl, c2_s2r_Wr, c2_r2r_Wl, c2_r2r_bl, c2_r2r_Wr,
           c2_s2s_Wl, c2_s2s_bl, c2_s2s_Wr,
           lin_W, lin_b):
    xp = jnp.pad(x_subject, ((0, N_PAD - NS), (0, 0)))
    npad = E_PAD - E
    pad_dst = NS + (jnp.arange(npad, dtype=jnp.int32) % (N_PAD - NS))
    srcs = jnp.concatenate(
        [ei_s2s[0], jnp.full((npad,), NS, jnp.int32)]).reshape(ROWS_W, K)
    dsts = jnp.concatenate([ei_s2s[1], pad_dst]).reshape(ROWS_W, K)
    z64 = jnp.zeros((N_PAD, H), _f32)
    z16 = jnp.zeros((N_PAD, CW), _f32)
    ones = jnp.ones((K, CW), _f32)
    bl1 = c1_s2s_bl.reshape(1, H)
    bl2 = c2_s2s_bl.reshape(1, H)
    wlin = jnp.pad(lin_W, ((0, 0), (0, 128 - O)))
    blin = jnp.pad(lin_b, ((0, 128 - O))).reshape(1, 128)

    y1, z1 = pl.pallas_call(
        _pre_body,
        out_shape=[jax.ShapeDtypeStruct((N_PAD, H), _f32)] * 2,
    )(xp, c1_s2s_Wl, c1_s2s_Wr)

    s1p, cntp = _seg_sum_kernel(True)(y1, srcs, dsts, z64, z16, ones)

    y2, z2 = pl.pallas_call(
        _mid_body,
        out_shape=[jax.ShapeDtypeStruct((N_PAD, H), _f32)] * 2,
    )(s1p, cntp, z1, bl1, c2_s2s_Wl, c2_s2s_Wr)

    s2p = _seg_sum_kernel(False)(y2, srcs, dsts, z64)

    outp = pl.pallas_call(
        _fin_body,
        out_shape=jax.ShapeDtypeStruct((N_PAD, 128), _f32),
    )(s2p, cntp, z2, bl2, wlin, blin)

    return outp[:NS, :O]


# speedup vs baseline: 1.1257x; 1.1247x over previous
import functools

import jax
import jax.numpy as jnp
from jax import lax
from jax.experimental import pallas as pl
from jax.experimental.pallas import tpu as pltpu
from jax.experimental.pallas import tpu_sc as plsc

NS = 10000
E = 640000
D = 128
H = 64
O = 2

NC = 2
NSUB = 16
NW = NC * NSUB
K = 128
NB = 8
G = 20
G0 = 30
G1 = 10
CH = NB * G
E_PAD = NW * K * CH
ROWS_W = E_PAD // K
N_PAD = 10016
CW = 16


def _seg_sum_kernel(with_count):
    mesh = plsc.VectorSubcoreMesh(core_axis_name="c", subcore_axis_name="s")
    out_type = [jax.ShapeDtypeStruct((NC, N_PAD, H), jnp.float32)]
    scratch = [
        pltpu.VMEM((NB, K), jnp.int32),
        pltpu.VMEM((NB, K), jnp.int32),
        pltpu.VMEM((NB, K), jnp.int32),
        pltpu.VMEM((NB, K), jnp.int32),
        pltpu.VMEM((NB * K, H), jnp.float32),
        pltpu.VMEM_SHARED((N_PAD, H), jnp.float32),
        pltpu.SemaphoreType.DMA,
        pltpu.SemaphoreType.DMA,
        pltpu.SemaphoreType.DMA,
        pltpu.SemaphoreType.DMA,
    ]
    if with_count:
        out_type.append(jax.ShapeDtypeStruct((NC, N_PAD, CW), jnp.float32))
        scratch += [
            pltpu.VMEM((K, CW), jnp.float32),
            pltpu.VMEM_SHARED((N_PAD, CW), jnp.float32),
        ]

    def body(*refs):
        if with_count:
            (y, srcs, dsts, z64, z16, ones_in, acc_out, cnt_out,
             sidxA, sidxB, didxA, didxB, rows, acc,
             semIA, semIB, semG, semS, ones_v, cnt) = refs
        else:
            (y, srcs, dsts, z64, acc_out,
             sidxA, sidxB, didxA, didxB, rows, acc,
             semIA, semIB, semG, semS) = refs
        cid = lax.axis_index("c")
        sid = lax.axis_index("s")

        @pl.when(sid == 0)
        def _init():
            pltpu.sync_copy(z64, acc)
            if with_count:
                pltpu.sync_copy(z16, cnt)

        if with_count:
            pltpu.sync_copy(ones_in, ones_v)
        plsc.subcore_barrier()

        row0 = jnp.where(cid == 0, sid * G0, NSUB * G0 + sid * G1) * NB
        npairs = jnp.where(cid == 0, G0 // 2, G1 // 2)

        def load_idx(grow, sidx, didx, sem):
            pltpu.async_copy(srcs.at[pl.ds(grow, NB)], sidx, sem)
            pltpu.async_copy(dsts.at[pl.ds(grow, NB)], didx, sem)

        def drain_idx(sidx, didx, sem):
            pltpu.make_async_copy(srcs.at[pl.ds(0, NB)], sidx, sem).wait()
            pltpu.make_async_copy(dsts.at[pl.ds(0, NB)], didx, sem).wait()

        def phase(sidx, didx, semI, prefetch_grow, prefetch_pred,
                  psidx, pdidx, psemI):
            drain_idx(sidx, didx, semI)
            gd = [
                pltpu.async_copy(
                    y.at[sidx.at[j]], rows.at[pl.ds(j * K, K)], semG)
                for j in range(NB)
            ]
            sd = []
            if with_count:
                sd += [
                    pltpu.async_copy(
                        ones_v, cnt.at[didx.at[j]], semS, add=True)
                    for j in range(NB)
                ]

            @pl.when(prefetch_pred)
            def _():
                load_idx(prefetch_grow, psidx, pdidx, psemI)

            for d in gd:
                d.wait()
            sd += [
                pltpu.async_copy(
                    rows.at[pl.ds(j * K, K)], acc.at[didx.at[j]],
                    semS, add=True)
                for j in range(NB)
            ]
            for d in sd:
                d.wait()

        load_idx(row0, sidxA, didxA, semIA)

        def group_pair(i, carry):
            g0row = row0 + (2 * i) * NB
            phase(sidxA, didxA, semIA,
                  g0row + NB, i >= 0, sidxB, didxB, semIB)
            phase(sidxB, didxB, semIB,
                  g0row + 2 * NB, i < npairs - 1, sidxA, didxA, semIA)
            return carry

        lax.fori_loop(0, npairs, group_pair, 0)
        plsc.subcore_barrier()

        @pl.when(sid == 0)
        def _export():
            pltpu.sync_copy(acc, acc_out.at[cid])
            if with_count:
                pltpu.sync_copy(cnt, cnt_out.at[cid])

    return pl.kernel(
        body,
        out_type=tuple(out_type) if with_count else out_type[0],
        mesh=mesh,
        scratch_types=scratch,
        compiler_params=pltpu.CompilerParams(use_tc_tiling_on_sc=False),
    )


def _pre_body(x_ref, wl_ref, wr_ref, y_ref, z_ref):
    x = x_ref[:]
    y_ref[:] = jnp.dot(x, wl_ref[:], preferred_element_type=jnp.float32)
    z_ref[:] = jnp.dot(x, wr_ref[:], preferred_element_type=jnp.float32)


def _mid_body(sp_ref, cp_ref, z_ref, bl_ref, wl_ref, wr_ref, y2_ref, z2_ref):
    s = sp_ref[0] + sp_ref[1]
    cnt = cp_ref[0, :, 0:1] + cp_ref[1, :, 0:1]
    mean = s / jnp.maximum(cnt, 1.0)
    h = jnp.maximum(mean + bl_ref[:] + z_ref[:], 0.0)
    y2_ref[:] = jnp.dot(h, wl_ref[:], preferred_element_type=jnp.float32)
    z2_ref[:] = jnp.dot(h, wr_ref[:], preferred_element_type=jnp.float32)


def _fin_body(sp_ref, cp_ref, z_ref, bl_ref, wlin_ref, blin_ref, out_ref):
    s = sp_ref[0] + sp_ref[1]
    cnt = cp_ref[0, :, 0:1] + cp_ref[1, :, 0:1]
    mean = s / jnp.maximum(cnt, 1.0)
    h = jnp.maximum(mean + bl_ref[:] + z_ref[:], 0.0)
    out_ref[:] = (
        jnp.dot(h, wlin_ref[:], preferred_element_type=jnp.float32)
        + blin_ref[:]
    )


_f32 = jnp.float32


def kernel(x_subject, x_roi, ei_s2r, ei_r2r, ei_s2s,
           c1_s2r_Wl, c1_s2r_bl, c1_s2r_Wr, c1_r2r_Wl, c1_r2r_bl, c1_r2r_Wr,
           c1_s2s_Wl, c1_s2s_bl, c1_s2s_Wr,
           c2_s2r_Wl, c2_s2r_bl, c2_s2r_Wr, c2_r2r_Wl, c2_r2r_bl, c2_r2r_Wr,
           c2_s2s_Wl, c2_s2s_bl, c2_s2s_Wr,
           lin_W, lin_b):
    xp = jnp.pad(x_subject, ((0, N_PAD - NS), (0, 0)))
    npad = E_PAD - E
    pad_dst = NS + (jnp.arange(npad, dtype=jnp.int32) % (N_PAD - NS))
    srcs = jnp.concatenate(
        [ei_s2s[0], jnp.full((npad,), NS, jnp.int32)]).reshape(ROWS_W, K)
    dsts = jnp.concatenate([ei_s2s[1], pad_dst]).reshape(ROWS_W, K)
    z64 = jnp.zeros((N_PAD, H), _f32)
    z16 = jnp.zeros((N_PAD, CW), _f32)
    ones = jnp.ones((K, CW), _f32)
    bl1 = c1_s2s_bl.reshape(1, H)
    bl2 = c2_s2s_bl.reshape(1, H)
    wlin = jnp.pad(lin_W, ((0, 0), (0, 128 - O)))
    blin = jnp.pad(lin_b, ((0, 128 - O))).reshape(1, 128)

    y1, z1 = pl.pallas_call(
        _pre_body,
        out_shape=[jax.ShapeDtypeStruct((N_PAD, H), _f32)] * 2,
    )(xp, c1_s2s_Wl, c1_s2s_Wr)

    s1p, cntp = _seg_sum_kernel(True)(y1, srcs, dsts, z64, z16, ones)

    y2, z2 = pl.pallas_call(
        _mid_body,
        out_shape=[jax.ShapeDtypeStruct((N_PAD, H), _f32)] * 2,
    )(s1p, cntp, z1, bl1, c2_s2s_Wl, c2_s2s_Wr)

    s2p = _seg_sum_kernel(False)(y2, srcs, dsts, z64)

    outp = pl.pallas_call(
        _fin_body,
        out_shape=jax.ShapeDtypeStruct((N_PAD, 128), _f32),
    )(s2p, cntp, z2, bl2, wlin, blin)

    return outp[:NS, :O]


# speedup vs baseline: 2.0478x; 1.8191x over previous
import functools

import jax
import jax.numpy as jnp
from jax import lax
from jax.experimental import pallas as pl
from jax.experimental.pallas import tpu as pltpu
from jax.experimental.pallas import tpu_sc as plsc

NS = 10000
E = 640000
D = 128
H = 64
O = 2

NC = 2
NSUB = 16
NW = NC * NSUB
K = 128
NB = 4
G = 40
CH = NB * G
E_PAD = NW * K * CH
ROWS_W = E_PAD // K
N_PAD = 10016
CW = 16


def _seg_sum_kernel(with_count):
    mesh = plsc.VectorSubcoreMesh(core_axis_name="c", subcore_axis_name="s")
    out_type = [jax.ShapeDtypeStruct((NC, N_PAD, H), jnp.float32)]
    scratch = [
        pltpu.VMEM((NB, K), jnp.int32),
        pltpu.VMEM((NB, K), jnp.int32),
        pltpu.VMEM((NB, K), jnp.int32),
        pltpu.VMEM((NB, K), jnp.int32),
        pltpu.VMEM((NB * K, H), jnp.float32),
        pltpu.VMEM_SHARED((N_PAD, H), jnp.float32),
        pltpu.SemaphoreType.DMA,
        pltpu.SemaphoreType.DMA,
        pltpu.SemaphoreType.DMA,
        pltpu.SemaphoreType.DMA,
    ]
    scratch.append(
        pltpu.VMEM_SHARED((N_PAD, H), jnp.float32))
    if with_count:
        out_type.append(jax.ShapeDtypeStruct((NC, N_PAD, CW), jnp.float32))
        scratch += [
            pltpu.VMEM((K, CW), jnp.float32),
            pltpu.VMEM_SHARED((N_PAD, CW), jnp.float32),
        ]

    def body(*refs):
        if with_count:
            (y, srcs, dsts, z64, z16, ones_in, acc_out, cnt_out,
             sidxA, sidxB, didxA, didxB, rows, acc,
             semIA, semIB, semG, semS, y_sh, ones_v, cnt) = refs
        else:
            (y, srcs, dsts, z64, acc_out,
             sidxA, sidxB, didxA, didxB, rows, acc,
             semIA, semIB, semG, semS, y_sh) = refs
        cid = lax.axis_index("c")
        sid = lax.axis_index("s")

        nrows = N_PAD // NSUB
        pltpu.sync_copy(y.at[pl.ds(sid * nrows, nrows)],
                        y_sh.at[pl.ds(sid * nrows, nrows)])

        @pl.when(sid == 0)
        def _init():
            pltpu.sync_copy(z64, acc)
            if with_count:
                pltpu.sync_copy(z16, cnt)

        if with_count:
            pltpu.sync_copy(ones_in, ones_v)
        plsc.subcore_barrier()

        row0 = (cid * NSUB + sid) * (G * NB)
        npairs = G // 2

        def load_idx(grow, sidx, didx, sem):
            pltpu.async_copy(srcs.at[pl.ds(grow, NB)], sidx, sem)
            pltpu.async_copy(dsts.at[pl.ds(grow, NB)], didx, sem)

        def drain_idx(sidx, didx, sem):
            pltpu.make_async_copy(srcs.at[pl.ds(0, NB)], sidx, sem).wait()
            pltpu.make_async_copy(dsts.at[pl.ds(0, NB)], didx, sem).wait()

        def phase(sidx, didx, semI, prefetch_grow, prefetch_pred,
                  psidx, pdidx, psemI):
            drain_idx(sidx, didx, semI)
            gd = [
                pltpu.async_copy(
                    y_sh.at[sidx.at[j]], rows.at[pl.ds(j * K, K)], semG)
                for j in range(NB)
            ]
            sd = []
            if with_count:
                sd += [
                    pltpu.async_copy(
                        ones_v, cnt.at[didx.at[j]], semS, add=True)
                    for j in range(NB)
                ]

            @pl.when(prefetch_pred)
            def _():
                load_idx(prefetch_grow, psidx, pdidx, psemI)

            for d in gd:
                d.wait()
            sd += [
                pltpu.async_copy(
                    rows.at[pl.ds(j * K, K)], acc.at[didx.at[j]],
                    semS, add=True)
                for j in range(NB)
            ]
            for d in sd:
                d.wait()

        load_idx(row0, sidxA, didxA, semIA)

        def group_pair(i, carry):
            g0row = row0 + (2 * i) * NB
            phase(sidxA, didxA, semIA,
                  g0row + NB, i >= 0, sidxB, didxB, semIB)
            phase(sidxB, didxB, semIB,
                  g0row + 2 * NB, i < npairs - 1, sidxA, didxA, semIA)
            return carry

        lax.fori_loop(0, npairs, group_pair, 0)
        plsc.subcore_barrier()

        @pl.when(sid == 0)
        def _export():
            pltpu.sync_copy(acc, acc_out.at[cid])
            if with_count:
                pltpu.sync_copy(cnt, cnt_out.at[cid])

    return pl.kernel(
        body,
        out_type=tuple(out_type) if with_count else out_type[0],
        mesh=mesh,
        scratch_types=scratch,
        compiler_params=pltpu.CompilerParams(use_tc_tiling_on_sc=False),
    )


def _pre_body(x_ref, wl_ref, wr_ref, y_ref, z_ref):
    x = x_ref[:]
    y_ref[:] = jnp.dot(x, wl_ref[:], preferred_element_type=jnp.float32)
    z_ref[:] = jnp.dot(x, wr_ref[:], preferred_element_type=jnp.float32)


def _mid_body(sp_ref, cp_ref, z_ref, bl_ref, wl_ref, wr_ref, y2_ref, z2_ref):
    s = sp_ref[0] + sp_ref[1]
    cnt = cp_ref[0, :, 0:1] + cp_ref[1, :, 0:1]
    mean = s / jnp.maximum(cnt, 1.0)
    h = jnp.maximum(mean + bl_ref[:] + z_ref[:], 0.0)
    y2_ref[:] = jnp.dot(h, wl_ref[:], preferred_element_type=jnp.float32)
    z2_ref[:] = jnp.dot(h, wr_ref[:], preferred_element_type=jnp.float32)


def _fin_body(sp_ref, cp_ref, z_ref, bl_ref, wlin_ref, blin_ref, out_ref):
    s = sp_ref[0] + sp_ref[1]
    cnt = cp_ref[0, :, 0:1] + cp_ref[1, :, 0:1]
    mean = s / jnp.maximum(cnt, 1.0)
    h = jnp.maximum(mean + bl_ref[:] + z_ref[:], 0.0)
    out_ref[:] = (
        jnp.dot(h, wlin_ref[:], preferred_element_type=jnp.float32)
        + blin_ref[:]
    )


_f32 = jnp.float32


def kernel(x_subject, x_roi, ei_s2r, ei_r2r, ei_s2s,
           c1_s2r_Wl, c1_s2r_bl, c1_s2r_Wr, c1_r2r_Wl, c1_r2r_bl, c1_r2r_Wr,
           c1_s2s_Wl, c1_s2s_bl, c1_s2s_Wr,
           c2_s2r_Wl, c2_s2r_bl, c2_s2r_Wr, c2_r2r_Wl, c2_r2r_bl, c2_r2r_Wr,
           c2_s2s_Wl, c2_s2s_bl, c2_s2s_Wr,
           lin_W, lin_b):
    xp = jnp.pad(x_subject, ((0, N_PAD - NS), (0, 0)))
    npad = E_PAD - E
    pad_dst = NS + (jnp.arange(npad, dtype=jnp.int32) % (N_PAD - NS))
    srcs = jnp.concatenate(
        [ei_s2s[0], jnp.full((npad,), NS, jnp.int32)]).reshape(ROWS_W, K)
    dsts = jnp.concatenate([ei_s2s[1], pad_dst]).reshape(ROWS_W, K)
    z64 = jnp.zeros((N_PAD, H), _f32)
    z16 = jnp.zeros((N_PAD, CW), _f32)
    ones = jnp.ones((K, CW), _f32)
    bl1 = c1_s2s_bl.reshape(1, H)
    bl2 = c2_s2s_bl.reshape(1, H)
    wlin = jnp.pad(lin_W, ((0, 0), (0, 128 - O)))
    blin = jnp.pad(lin_b, ((0, 128 - O))).reshape(1, 128)

    y1, z1 = pl.pallas_call(
        _pre_body,
        out_shape=[jax.ShapeDtypeStruct((N_PAD, H), _f32)] * 2,
    )(xp, c1_s2s_Wl, c1_s2s_Wr)

    s1p, cntp = _seg_sum_kernel(True)(y1, srcs, dsts, z64, z16, ones)

    y2, z2 = pl.pallas_call(
        _mid_body,
        out_shape=[jax.ShapeDtypeStruct((N_PAD, H), _f32)] * 2,
    )(s1p, cntp, z1, bl1, c2_s2s_Wl, c2_s2s_Wr)

    s2p = _seg_sum_kernel(False)(y2, srcs, dsts, z64)

    outp = pl.pallas_call(
        _fin_body,
        out_shape=jax.ShapeDtypeStruct((N_PAD, 128), _f32),
    )(s2p, cntp, z2, bl2, wlin, blin)

    return outp[:NS, :O]


# speedup vs baseline: 2.1872x; 1.0681x over previous
import functools

import jax
import jax.numpy as jnp
from jax import lax
from jax.experimental import pallas as pl
from jax.experimental.pallas import tpu as pltpu
from jax.experimental.pallas import tpu_sc as plsc

NS = 10000
E = 640000
D = 128
H = 64
O = 2

NC = 2
NSUB = 16
NW = NC * NSUB
K = 128
NB = 2
G = 80
ITERS = G // 4
CH = NB * G
E_PAD = NW * K * CH
ROWS_W = E_PAD // K
N_PAD = 10016
CW = 16


def _seg_sum_kernel(with_count):
    mesh = plsc.VectorSubcoreMesh(core_axis_name="c", subcore_axis_name="s")
    out_type = [jax.ShapeDtypeStruct((NC, N_PAD, H), jnp.float32)]
    scratch = (
        [pltpu.VMEM((NB, K), jnp.int32) for _ in range(4)]
        + [pltpu.VMEM((NB, K), jnp.int32) for _ in range(4)]
        + [pltpu.VMEM((NB * K, H), jnp.float32) for _ in range(2)]
        + [pltpu.VMEM_SHARED((N_PAD, H), jnp.float32),
           pltpu.VMEM_SHARED((N_PAD, H), jnp.float32)]
        + [pltpu.SemaphoreType.DMA] * 7
    )
    if with_count:
        out_type.append(jax.ShapeDtypeStruct((NC, N_PAD, CW), jnp.float32))
        scratch += [
            pltpu.VMEM((K, CW), jnp.float32),
            pltpu.VMEM_SHARED((N_PAD, CW), jnp.float32),
        ]

    def body(*refs):
        if with_count:
            (y, srcs, dsts, z64, z16, ones_in, acc_out, cnt_out,
             *rest) = refs
        else:
            (y, srcs, dsts, z64, acc_out, *rest) = refs
        (s0, s1, s2, s3, d0, d1, d2, d3, rows0, rows1, acc, y_sh,
         i0, i1, i2, i3, ss0, ss1, semG, *restc) = rest
        if with_count:
            ones_v, cnt = restc
        sidx = [s0, s1, s2, s3]
        didx = [d0, d1, d2, d3]
        rows = [rows0, rows1]
        semI = [i0, i1, i2, i3]
        semS = [ss0, ss1]

        cid = lax.axis_index("c")
        sid = lax.axis_index("s")

        nrows = N_PAD // NSUB
        pltpu.sync_copy(y.at[pl.ds(sid * nrows, nrows)],
                        y_sh.at[pl.ds(sid * nrows, nrows)])

        @pl.when(sid == 0)
        def _init():
            pltpu.sync_copy(z64, acc)
            if with_count:
                pltpu.sync_copy(z16, cnt)

        if with_count:
            pltpu.sync_copy(ones_in, ones_v)
        plsc.subcore_barrier()

        row0 = (cid * NSUB + sid) * (G * NB)

        def drain_scatters(r2):
            for j in range(NB):
                pltpu.make_async_copy(
                    rows[r2].at[pl.ds(j * K, K)], acc.at[pl.ds(0, K)],
                    semS[r2]).wait()
                if with_count:
                    pltpu.make_async_copy(
                        ones_v, cnt.at[pl.ds(0, K)], semS[r2]).wait()

        def load_idx(grow, r):
            pltpu.async_copy(srcs.at[pl.ds(grow, NB)], sidx[r], semI[r])
            pltpu.async_copy(dsts.at[pl.ds(grow, NB)], didx[r], semI[r])

        def phase(i, r, maybe_first):
            r2 = r % 2
            g = i * 4 + r
            if maybe_first:
                @pl.when(i > 0)
                def _():
                    drain_scatters(r2)
            else:
                drain_scatters(r2)
            pltpu.make_async_copy(srcs.at[pl.ds(0, NB)], sidx[r],
                                  semI[r]).wait()
            pltpu.make_async_copy(dsts.at[pl.ds(0, NB)], didx[r],
                                  semI[r]).wait()
            gd = [
                pltpu.async_copy(y_sh.at[sidx[r].at[j]],
                                 rows[r2].at[pl.ds(j * K, K)], semG)
                for j in range(NB)
            ]
            rp = (r + 2) % 4
            prow = row0 + (g + 2) * NB

            if r >= 2:
                @pl.when(i < ITERS - 1)
                def _():
                    load_idx(prow, rp)
            else:
                load_idx(prow, rp)
            for dsc in gd:
                dsc.wait()
            for j in range(NB):
                pltpu.async_copy(rows[r2].at[pl.ds(j * K, K)],
                                 acc.at[didx[r].at[j]], semS[r2], add=True)
                if with_count:
                    pltpu.async_copy(ones_v, cnt.at[didx[r].at[j]],
                                     semS[r2], add=True)

        load_idx(row0, 0)
        load_idx(row0 + NB, 1)

        def loop_body(i, carry):
            phase(i, 0, True)
            phase(i, 1, True)
            phase(i, 2, False)
            phase(i, 3, False)
            return carry

        lax.fori_loop(0, ITERS, loop_body, 0)
        drain_scatters(0)
        drain_scatters(1)
        plsc.subcore_barrier()

        @pl.when(sid == 0)
        def _export():
            pltpu.sync_copy(acc, acc_out.at[cid])
            if with_count:
                pltpu.sync_copy(cnt, cnt_out.at[cid])

    return pl.kernel(
        body,
        out_type=tuple(out_type) if with_count else out_type[0],
        mesh=mesh,
        scratch_types=scratch,
        compiler_params=pltpu.CompilerParams(use_tc_tiling_on_sc=False),
    )


def _pre_body(x_ref, wl_ref, wr_ref, y_ref, z_ref):
    x = x_ref[:]
    y_ref[:] = jnp.dot(x, wl_ref[:], preferred_element_type=jnp.float32)
    z_ref[:] = jnp.dot(x, wr_ref[:], preferred_element_type=jnp.float32)


def _mid_body(sp_ref, cp_ref, z_ref, bl_ref, wl_ref, wr_ref, y2_ref, z2_ref):
    s = sp_ref[0] + sp_ref[1]
    cnt = cp_ref[0, :, 0:1] + cp_ref[1, :, 0:1]
    mean = s / jnp.maximum(cnt, 1.0)
    h = jnp.maximum(mean + bl_ref[:] + z_ref[:], 0.0)
    y2_ref[:] = jnp.dot(h, wl_ref[:], preferred_element_type=jnp.float32)
    z2_ref[:] = jnp.dot(h, wr_ref[:], preferred_element_type=jnp.float32)


def _fin_body(sp_ref, cp_ref, z_ref, bl_ref, wlin_ref, blin_ref, out_ref):
    s = sp_ref[0] + sp_ref[1]
    cnt = cp_ref[0, :, 0:1] + cp_ref[1, :, 0:1]
    mean = s / jnp.maximum(cnt, 1.0)
    h = jnp.maximum(mean + bl_ref[:] + z_ref[:], 0.0)
    out_ref[:] = (
        jnp.dot(h, wlin_ref[:], preferred_element_type=jnp.float32)
        + blin_ref[:]
    )


_f32 = jnp.float32


def kernel(x_subject, x_roi, ei_s2r, ei_r2r, ei_s2s,
           c1_s2r_Wl, c1_s2r_bl, c1_s2r_Wr, c1_r2r_Wl, c1_r2r_bl, c1_r2r_Wr,
           c1_s2s_Wl, c1_s2s_bl, c1_s2s_Wr,
           c2_s2r_Wl, c2_s2r_bl, c2_s2r_Wr, c2_r2r_Wl, c2_r2r_bl, c2_r2r_Wr,
           c2_s2s_Wl, c2_s2s_bl, c2_s2s_Wr,
           lin_W, lin_b):
    xp = jnp.pad(x_subject, ((0, N_PAD - NS), (0, 0)))
    npad = E_PAD - E
    pad_dst = NS + (jnp.arange(npad, dtype=jnp.int32) % (N_PAD - NS))
    srcs = jnp.concatenate(
        [ei_s2s[0], jnp.full((npad,), NS, jnp.int32)]).reshape(ROWS_W, K)
    dsts = jnp.concatenate([ei_s2s[1], pad_dst]).reshape(ROWS_W, K)
    z64 = jnp.zeros((N_PAD, H), _f32)
    z16 = jnp.zeros((N_PAD, CW), _f32)
    ones = jnp.ones((K, CW), _f32)
    bl1 = c1_s2s_bl.reshape(1, H)
    bl2 = c2_s2s_bl.reshape(1, H)
    wlin = jnp.pad(lin_W, ((0, 0), (0, 128 - O)))
    blin = jnp.pad(lin_b, ((0, 128 - O))).reshape(1, 128)

    y1, z1 = pl.pallas_call(
        _pre_body,
        out_shape=[jax.ShapeDtypeStruct((N_PAD, H), _f32)] * 2,
    )(xp, c1_s2s_Wl, c1_s2s_Wr)

    s1p, cntp = _seg_sum_kernel(True)(y1, srcs, dsts, z64, z16, ones)

    y2, z2 = pl.pallas_call(
        _mid_body,
        out_shape=[jax.ShapeDtypeStruct((N_PAD, H), _f32)] * 2,
    )(s1p, cntp, z1, bl1, c2_s2s_Wl, c2_s2s_Wr)

    s2p = _seg_sum_kernel(False)(y2, srcs, dsts, z64)

    outp = pl.pallas_call(
        _fin_body,
        out_shape=jax.ShapeDtypeStruct((N_PAD, 128), _f32),
    )(s2p, cntp, z2, bl2, wlin, blin)

    return outp[:NS, :O]


# speedup vs baseline: 2.3996x; 1.0971x over previous
import functools

import jax
import jax.numpy as jnp
from jax import lax
from jax.experimental import pallas as pl
from jax.experimental.pallas import tpu as pltpu
from jax.experimental.pallas import tpu_sc as plsc

NS = 10000
E = 640000
D = 128
H = 64
O = 2

NC = 2
NSUB = 16
NW = NC * NSUB
K = 128
NB = 2
G = 80
ITERS = G // 4
CH = NB * G
E_PAD = NW * K * CH
ROWS_W = E_PAD // K
N_PAD = 10016
WC = 72


def _seg_sum_kernel(width):
    mesh = plsc.VectorSubcoreMesh(core_axis_name="c", subcore_axis_name="s")
    out_type = jax.ShapeDtypeStruct((NC, N_PAD, width), jnp.float32)
    scratch = (
        [pltpu.VMEM((NB, K), jnp.int32) for _ in range(4)]
        + [pltpu.VMEM((NB, K), jnp.int32) for _ in range(4)]
        + [pltpu.VMEM((NB * K, width), jnp.float32) for _ in range(2)]
        + [pltpu.VMEM_SHARED((N_PAD, width), jnp.float32),
           pltpu.VMEM_SHARED((N_PAD, width), jnp.float32)]
        + [pltpu.SemaphoreType.DMA] * 7
    )

    def body(*refs):
        (y, srcs, dsts, zeros, acc_out,
         s0, s1, s2, s3, d0, d1, d2, d3, rows0, rows1, acc, y_sh,
         i0, i1, i2, i3, ss0, ss1, semG) = refs
        sidx = [s0, s1, s2, s3]
        didx = [d0, d1, d2, d3]
        rows = [rows0, rows1]
        semI = [i0, i1, i2, i3]
        semS = [ss0, ss1]

        cid = lax.axis_index("c")
        sid = lax.axis_index("s")

        nrows = N_PAD // NSUB
        pltpu.sync_copy(y.at[pl.ds(sid * nrows, nrows)],
                        y_sh.at[pl.ds(sid * nrows, nrows)])

        @pl.when(sid == 0)
        def _init():
            pltpu.sync_copy(zeros, acc)

        plsc.subcore_barrier()

        row0 = (cid * NSUB + sid) * (G * NB)

        def drain_scatters(r2):
            for j in range(NB):
                pltpu.make_async_copy(
                    rows[r2].at[pl.ds(j * K, K)], acc.at[pl.ds(0, K)],
                    semS[r2]).wait()

        def load_idx(grow, r):
            pltpu.async_copy(srcs.at[pl.ds(grow, NB)], sidx[r], semI[r])
            pltpu.async_copy(dsts.at[pl.ds(grow, NB)], didx[r], semI[r])

        def phase(i, r, maybe_first):
            r2 = r % 2
            g = i * 4 + r
            if maybe_first:
                @pl.when(i > 0)
                def _():
                    drain_scatters(r2)
            else:
                drain_scatters(r2)
            pltpu.make_async_copy(srcs.at[pl.ds(0, NB)], sidx[r],
                                  semI[r]).wait()
            pltpu.make_async_copy(dsts.at[pl.ds(0, NB)], didx[r],
                                  semI[r]).wait()
            gd = [
                pltpu.async_copy(y_sh.at[sidx[r].at[j]],
                                 rows[r2].at[pl.ds(j * K, K)], semG)
                for j in range(NB)
            ]
            rp = (r + 2) % 4
            prow = row0 + (g + 2) * NB

            if r >= 2:
                @pl.when(i < ITERS - 1)
                def _():
                    load_idx(prow, rp)
            else:
                load_idx(prow, rp)
            for dsc in gd:
                dsc.wait()
            for j in range(NB):
                pltpu.async_copy(rows[r2].at[pl.ds(j * K, K)],
                                 acc.at[didx[r].at[j]], semS[r2], add=True)

        load_idx(row0, 0)
        load_idx(row0 + NB, 1)

        def loop_body(i, carry):
            phase(i, 0, True)
            phase(i, 1, True)
            phase(i, 2, False)
            phase(i, 3, False)
            return carry

        lax.fori_loop(0, ITERS, loop_body, 0)
        drain_scatters(0)
        drain_scatters(1)
        plsc.subcore_barrier()

        @pl.when(sid == 0)
        def _export():
            pltpu.sync_copy(acc, acc_out.at[cid])

    return pl.kernel(
        body,
        out_type=out_type,
        mesh=mesh,
        scratch_types=scratch,
        compiler_params=pltpu.CompilerParams(use_tc_tiling_on_sc=False),
    )


def _pre_body(x_ref, wl_ref, wr_ref, e_ref, y_ref, z_ref):
    x = x_ref[:]
    yl = jnp.dot(x, wl_ref[:], preferred_element_type=jnp.float32)
    e = jnp.broadcast_to(e_ref[:], (N_PAD, WC - H))
    y_ref[:] = jnp.concatenate([yl, e], axis=1)
    z_ref[:] = jnp.dot(x, wr_ref[:], preferred_element_type=jnp.float32)


def _mid_body(sp_ref, z_ref, bl_ref, wl_ref, wr_ref, y2_ref, z2_ref, c_ref):
    s = sp_ref[0] + sp_ref[1]
    cnt = s[:, H:H + 1]
    mean = s[:, :H] / jnp.maximum(cnt, 1.0)
    h = jnp.maximum(mean + bl_ref[:] + z_ref[:], 0.0)
    y2_ref[:] = jnp.dot(h, wl_ref[:], preferred_element_type=jnp.float32)
    z2_ref[:] = jnp.dot(h, wr_ref[:], preferred_element_type=jnp.float32)
    c_ref[:] = jnp.broadcast_to(cnt, (N_PAD, 8))


def _fin_body(sp_ref, c_ref, z_ref, bl_ref, wlin_ref, blin_ref, out_ref):
    s = sp_ref[0] + sp_ref[1]
    cnt = c_ref[:, 0:1]
    mean = s / jnp.maximum(cnt, 1.0)
    h = jnp.maximum(mean + bl_ref[:] + z_ref[:], 0.0)
    out_ref[:] = (
        jnp.dot(h, wlin_ref[:], preferred_element_type=jnp.float32)
        + blin_ref[:]
    )


_f32 = jnp.float32


def kernel(x_subject, x_roi, ei_s2r, ei_r2r, ei_s2s,
           c1_s2r_Wl, c1_s2r_bl, c1_s2r_Wr, c1_r2r_Wl, c1_r2r_bl, c1_r2r_Wr,
           c1_s2s_Wl, c1_s2s_bl, c1_s2s_Wr,
           c2_s2r_Wl, c2_s2r_bl, c2_s2r_Wr, c2_r2r_Wl, c2_r2r_bl, c2_r2r_Wr,
           c2_s2s_Wl, c2_s2s_bl, c2_s2s_Wr,
           lin_W, lin_b):
    xp = jnp.pad(x_subject, ((0, N_PAD - NS), (0, 0)))
    npad = E_PAD - E
    pad_dst = NS + (jnp.arange(npad, dtype=jnp.int32) % (N_PAD - NS))
    srcs = jnp.concatenate(
        [ei_s2s[0], jnp.full((npad,), NS, jnp.int32)]).reshape(ROWS_W, K)
    dsts = jnp.concatenate([ei_s2s[1], pad_dst]).reshape(ROWS_W, K)
    z72 = jnp.zeros((N_PAD, WC), _f32)
    z64 = jnp.zeros((N_PAD, H), _f32)
    ecol = jnp.pad(jnp.ones((1, 1), _f32), ((0, 0), (0, WC - H - 1)))
    bl1 = c1_s2s_bl.reshape(1, H)
    bl2 = c2_s2s_bl.reshape(1, H)
    wlin = jnp.pad(lin_W, ((0, 0), (0, 128 - O)))
    blin = jnp.pad(lin_b, ((0, 128 - O))).reshape(1, 128)

    y1, z1 = pl.pallas_call(
        _pre_body,
        out_shape=[jax.ShapeDtypeStruct((N_PAD, WC), _f32),
                   jax.ShapeDtypeStruct((N_PAD, H), _f32)],
    )(xp, c1_s2s_Wl, c1_s2s_Wr, ecol)

    s1p = _seg_sum_kernel(WC)(y1, srcs, dsts, z72)

    y2, z2, cntc = pl.pallas_call(
        _mid_body,
        out_shape=[jax.ShapeDtypeStruct((N_PAD, H), _f32),
                   jax.ShapeDtypeStruct((N_PAD, H), _f32),
                   jax.ShapeDtypeStruct((N_PAD, 8), _f32)],
    )(s1p, z1, bl1, c2_s2s_Wl, c2_s2s_Wr)

    s2p = _seg_sum_kernel(H)(y2, srcs, dsts, z64)

    outp = pl.pallas_call(
        _fin_body,
        out_shape=jax.ShapeDtypeStruct((N_PAD, 128), _f32),
    )(s2p, cntc, z2, bl2, wlin, blin)

    return outp[:NS, :O]


# speedup vs baseline: 2.5010x; 1.0422x over previous
import functools

import jax
import jax.numpy as jnp
from jax import lax
from jax.experimental import pallas as pl
from jax.experimental.pallas import tpu as pltpu
from jax.experimental.pallas import tpu_sc as plsc

NS = 10000
E = 640000
D = 128
H = 64
O = 2

NC = 2
NSUB = 16
NW = NC * NSUB
K = 128
NB = 2
G = 80
ITERS = G // 4
CH = NB * G
E_PAD = NW * K * CH
ROWS_W = E_PAD // K
N_PAD = 10016
WC = 72


def _seg_sum_kernel(width):
    mesh = plsc.VectorSubcoreMesh(core_axis_name="c", subcore_axis_name="s")
    out_type = jax.ShapeDtypeStruct((NC, N_PAD, width), jnp.float32)
    scratch = (
        [pltpu.VMEM((NB, K), jnp.int32) for _ in range(4)]
        + [pltpu.VMEM((NB, K), jnp.int32) for _ in range(4)]
        + [pltpu.VMEM((NB * K, width), jnp.float32) for _ in range(2)]
        + [pltpu.VMEM_SHARED((N_PAD, width), jnp.float32),
           pltpu.VMEM_SHARED((N_PAD, width), jnp.float32)]
        + [pltpu.SemaphoreType.DMA] * 7
    )

    def body(*refs):
        (y, srcs, dsts, zeros, acc_out,
         s0, s1, s2, s3, d0, d1, d2, d3, rows0, rows1, acc, y_sh,
         i0, i1, i2, i3, ss0, ss1, semG) = refs
        sidx = [s0, s1, s2, s3]
        didx = [d0, d1, d2, d3]
        rows = [rows0, rows1]
        semI = [i0, i1, i2, i3]
        semS = [ss0, ss1]

        cid = lax.axis_index("c")
        sid = lax.axis_index("s")

        nrows = N_PAD // NSUB
        tsl = pl.ds(sid * nrows, nrows)
        pltpu.async_copy(y.at[tsl], y_sh.at[tsl], semG)
        pltpu.async_copy(zeros.at[tsl], acc.at[tsl], semG)
        pltpu.make_async_copy(y.at[tsl], y_sh.at[tsl], semG).wait()
        pltpu.make_async_copy(zeros.at[tsl], acc.at[tsl], semG).wait()
        plsc.subcore_barrier()

        row0 = (cid * NSUB + sid) * (G * NB)

        def drain_scatters(r2):
            for j in range(NB):
                pltpu.make_async_copy(
                    rows[r2].at[pl.ds(j * K, K)], acc.at[pl.ds(0, K)],
                    semS[r2]).wait()

        def load_idx(grow, r):
            pltpu.async_copy(srcs.at[pl.ds(grow, NB)], sidx[r], semI[r])
            pltpu.async_copy(dsts.at[pl.ds(grow, NB)], didx[r], semI[r])

        def phase(i, r, maybe_first):
            r2 = r % 2
            g = i * 4 + r
            if maybe_first:
                @pl.when(i > 0)
                def _():
                    drain_scatters(r2)
            else:
                drain_scatters(r2)
            pltpu.make_async_copy(srcs.at[pl.ds(0, NB)], sidx[r],
                                  semI[r]).wait()
            pltpu.make_async_copy(dsts.at[pl.ds(0, NB)], didx[r],
                                  semI[r]).wait()
            gd = [
                pltpu.async_copy(y_sh.at[sidx[r].at[j]],
                                 rows[r2].at[pl.ds(j * K, K)], semG)
                for j in range(NB)
            ]
            rp = (r + 2) % 4
            prow = row0 + (g + 2) * NB

            if r >= 2:
                @pl.when(i < ITERS - 1)
                def _():
                    load_idx(prow, rp)
            else:
                load_idx(prow, rp)
            for dsc in gd:
                dsc.wait()
            for j in range(NB):
                pltpu.async_copy(rows[r2].at[pl.ds(j * K, K)],
                                 acc.at[didx[r].at[j]], semS[r2], add=True)

        load_idx(row0, 0)
        load_idx(row0 + NB, 1)

        def loop_body(i, carry):
            phase(i, 0, True)
            phase(i, 1, True)
            phase(i, 2, False)
            phase(i, 3, False)
            return carry

        lax.fori_loop(0, ITERS, loop_body, 0)
        drain_scatters(0)
        drain_scatters(1)
        plsc.subcore_barrier()
        pltpu.sync_copy(acc.at[tsl], acc_out.at[cid].at[tsl])

    return pl.kernel(
        body,
        out_type=out_type,
        mesh=mesh,
        scratch_types=scratch,
        compiler_params=pltpu.CompilerParams(use_tc_tiling_on_sc=False),
    )


def _pre_body(x_ref, wl_ref, wr_ref, e_ref, y_ref, z_ref):
    x = x_ref[:]
    yl = jnp.dot(x, wl_ref[:], preferred_element_type=jnp.float32)
    e = jnp.broadcast_to(e_ref[:], (N_PAD, WC - H))
    y_ref[:] = jnp.concatenate([yl, e], axis=1)
    z_ref[:] = jnp.dot(x, wr_ref[:], preferred_element_type=jnp.float32)


def _mid_body(sp_ref, z_ref, bl_ref, wl_ref, wr_ref, y2_ref, z2_ref, c_ref):
    s = sp_ref[0] + sp_ref[1]
    cnt = s[:, H:H + 1]
    mean = s[:, :H] / jnp.maximum(cnt, 1.0)
    h = jnp.maximum(mean + bl_ref[:] + z_ref[:], 0.0)
    y2_ref[:] = jnp.dot(h, wl_ref[:], preferred_element_type=jnp.float32)
    z2_ref[:] = jnp.dot(h, wr_ref[:], preferred_element_type=jnp.float32)
    c_ref[:] = jnp.broadcast_to(cnt, (N_PAD, 8))


def _fin_body(sp_ref, c_ref, z_ref, bl_ref, wlin_ref, blin_ref, out_ref):
    s = sp_ref[0] + sp_ref[1]
    cnt = c_ref[:, 0:1]
    mean = s / jnp.maximum(cnt, 1.0)
    h = jnp.maximum(mean + bl_ref[:] + z_ref[:], 0.0)
    out_ref[:] = (
        jnp.dot(h, wlin_ref[:], preferred_element_type=jnp.float32)
        + blin_ref[:]
    )


_f32 = jnp.float32


def kernel(x_subject, x_roi, ei_s2r, ei_r2r, ei_s2s,
           c1_s2r_Wl, c1_s2r_bl, c1_s2r_Wr, c1_r2r_Wl, c1_r2r_bl, c1_r2r_Wr,
           c1_s2s_Wl, c1_s2s_bl, c1_s2s_Wr,
           c2_s2r_Wl, c2_s2r_bl, c2_s2r_Wr, c2_r2r_Wl, c2_r2r_bl, c2_r2r_Wr,
           c2_s2s_Wl, c2_s2s_bl, c2_s2s_Wr,
           lin_W, lin_b):
    xp = jnp.pad(x_subject, ((0, N_PAD - NS), (0, 0)))
    npad = E_PAD - E
    pad_col = jnp.full((npad,), NS, jnp.int32)
    srcs = jnp.concatenate([ei_s2s[0], pad_col]).reshape(ROWS_W, K)
    dsts = jnp.concatenate([ei_s2s[1], pad_col]).reshape(ROWS_W, K)
    z72 = jnp.zeros((N_PAD, WC), _f32)
    z64 = jnp.zeros((N_PAD, H), _f32)
    ecol = jnp.pad(jnp.ones((1, 1), _f32), ((0, 0), (0, WC - H - 1)))
    bl1 = c1_s2s_bl.reshape(1, H)
    bl2 = c2_s2s_bl.reshape(1, H)
    wlin = jnp.pad(lin_W, ((0, 0), (0, 128 - O)))
    blin = jnp.pad(lin_b, ((0, 128 - O))).reshape(1, 128)

    y1, z1 = pl.pallas_call(
        _pre_body,
        out_shape=[jax.ShapeDtypeStruct((N_PAD, WC), _f32),
                   jax.ShapeDtypeStruct((N_PAD, H), _f32)],
    )(xp, c1_s2s_Wl, c1_s2s_Wr, ecol)

    s1p = _seg_sum_kernel(WC)(y1, srcs, dsts, z72)

    y2, z2, cntc = pl.pallas_call(
        _mid_body,
        out_shape=[jax.ShapeDtypeStruct((N_PAD, H), _f32),
                   jax.ShapeDtypeStruct((N_PAD, H), _f32),
                   jax.ShapeDtypeStruct((N_PAD, 8), _f32)],
    )(s1p, z1, bl1, c2_s2s_Wl, c2_s2s_Wr)

    s2p = _seg_sum_kernel(H)(y2, srcs, dsts, z64)

    outp = pl.pallas_call(
        _fin_body,
        out_shape=jax.ShapeDtypeStruct((N_PAD, 128), _f32),
    )(s2p, cntc, z2, bl2, wlin, blin)

    return outp[:NS, :O]


# speedup vs baseline: 2.5027x; 1.0007x over previous
import jax
import jax.numpy as jnp
from jax import lax
from jax.experimental import pallas as pl
from jax.experimental.pallas import tpu as pltpu
from jax.experimental.pallas import tpu_sc as plsc

NS = 10000
E = 640000
D = 128
H = 64
O = 2

NC = 2
NSUB = 16
NW = NC * NSUB
K = 128
NB = 2
G = 80
ITERS = G // 4
CH = NB * G
E_PAD = NW * K * CH
ROWS_W = E_PAD // K
N_PAD = 10016
WC = 72


def _seg_sum_kernel(width):
    mesh = plsc.VectorSubcoreMesh(core_axis_name="c", subcore_axis_name="s")
    out_type = jax.ShapeDtypeStruct((NC, N_PAD, width), jnp.float32)
    scratch = (
        [pltpu.VMEM((NB, K), jnp.int32) for _ in range(4)]
        + [pltpu.VMEM((NB, K), jnp.int32) for _ in range(4)]
        + [pltpu.VMEM((NB * K, width), jnp.float32) for _ in range(2)]
        + [pltpu.VMEM_SHARED((N_PAD, width), jnp.float32),
           pltpu.VMEM_SHARED((N_PAD, width), jnp.float32)]
        + [pltpu.SemaphoreType.DMA] * 7
    )

    def body(*refs):
        (y, srcs, dsts, zeros, acc_out,
         s0, s1, s2, s3, d0, d1, d2, d3, rows0, rows1, acc, y_sh,
         i0, i1, i2, i3, ss0, ss1, semG) = refs
        sidx = [s0, s1, s2, s3]
        didx = [d0, d1, d2, d3]
        rows = [rows0, rows1]
        semI = [i0, i1, i2, i3]
        semS = [ss0, ss1]

        cid = lax.axis_index("c")
        sid = lax.axis_index("s")

        nrows = N_PAD // NSUB
        tsl = pl.ds(sid * nrows, nrows)
        pltpu.async_copy(y.at[tsl], y_sh.at[tsl], semG)
        pltpu.async_copy(zeros.at[tsl], acc.at[tsl], semG)
        pltpu.make_async_copy(y.at[tsl], y_sh.at[tsl], semG).wait()
        pltpu.make_async_copy(zeros.at[tsl], acc.at[tsl], semG).wait()
        plsc.subcore_barrier()

        row0 = (cid * NSUB + sid) * (G * NB)

        def drain_scatters(r2):
            for j in range(NB):
                pltpu.make_async_copy(
                    rows[r2].at[pl.ds(j * K, K)], acc.at[pl.ds(0, K)],
                    semS[r2]).wait()

        def load_idx(grow, r):
            pltpu.async_copy(srcs.at[pl.ds(grow, NB)], sidx[r], semI[r])
            pltpu.async_copy(dsts.at[pl.ds(grow, NB)], didx[r], semI[r])

        def phase(i, r, maybe_first):
            r2 = r % 2
            g = i * 4 + r
            if maybe_first:
                @pl.when(i > 0)
                def _():
                    drain_scatters(r2)
            else:
                drain_scatters(r2)
            pltpu.make_async_copy(srcs.at[pl.ds(0, NB)], sidx[r],
                                  semI[r]).wait()
            pltpu.make_async_copy(dsts.at[pl.ds(0, NB)], didx[r],
                                  semI[r]).wait()
            gd = [
                pltpu.async_copy(y_sh.at[sidx[r].at[j]],
                                 rows[r2].at[pl.ds(j * K, K)], semG)
                for j in range(NB)
            ]
            rp = (r + 2) % 4
            prow = row0 + (g + 2) * NB

            if r >= 2:
                @pl.when(i < ITERS - 1)
                def _():
                    load_idx(prow, rp)
            else:
                load_idx(prow, rp)
            for dsc in gd:
                dsc.wait()
            for j in range(NB):
                pltpu.async_copy(rows[r2].at[pl.ds(j * K, K)],
                                 acc.at[didx[r].at[j]], semS[r2], add=True)

        load_idx(row0, 0)
        load_idx(row0 + NB, 1)

        def loop_body(i, carry):
            phase(i, 0, True)
            phase(i, 1, True)
            phase(i, 2, False)
            phase(i, 3, False)
            return carry

        lax.fori_loop(0, ITERS, loop_body, 0)
        drain_scatters(0)
        drain_scatters(1)
        plsc.subcore_barrier()
        pltpu.sync_copy(acc.at[tsl], acc_out.at[cid].at[tsl])

    return pl.kernel(
        body,
        out_type=out_type,
        mesh=mesh,
        scratch_types=scratch,
        compiler_params=pltpu.CompilerParams(use_tc_tiling_on_sc=False),
    )


def _pre_body(x_ref, wl_ref, wr_ref, e_ref, y_ref, z_ref):
    x = x_ref[:]
    yl = jnp.dot(x, wl_ref[:], preferred_element_type=jnp.float32)
    e = jnp.broadcast_to(e_ref[:], (N_PAD, WC - H))
    y_ref[:] = jnp.concatenate([yl, e], axis=1)
    z_ref[:] = jnp.dot(x, wr_ref[:], preferred_element_type=jnp.float32)


def _mid_body(sp_ref, z_ref, bl_ref, wl_ref, wr_ref, y2_ref, z2_ref, c_ref):
    s = sp_ref[0] + sp_ref[1]
    cnt = s[:, H:H + 1]
    mean = s[:, :H] / jnp.maximum(cnt, 1.0)
    h = jnp.maximum(mean + bl_ref[:] + z_ref[:], 0.0)
    y2_ref[:] = jnp.dot(h, wl_ref[:], preferred_element_type=jnp.float32)
    z2_ref[:] = jnp.dot(h, wr_ref[:], preferred_element_type=jnp.float32)
    c_ref[:] = jnp.broadcast_to(cnt, (N_PAD, 8))


def _fin_body(sp_ref, c_ref, z_ref, bl_ref, wlin_ref, blin_ref, out_ref):
    s = sp_ref[0] + sp_ref[1]
    cnt = c_ref[:, 0:1]
    mean = s / jnp.maximum(cnt, 1.0)
    h = jnp.maximum(mean + bl_ref[:] + z_ref[:], 0.0)
    out_ref[:] = (
        jnp.dot(h, wlin_ref[:], preferred_element_type=jnp.float32)
        + blin_ref[:]
    )


_f32 = jnp.float32


def kernel(x_subject, x_roi, ei_s2r, ei_r2r, ei_s2s,
           c1_s2r_Wl, c1_s2r_bl, c1_s2r_Wr, c1_r2r_Wl, c1_r2r_bl, c1_r2r_Wr,
           c1_s2s_Wl, c1_s2s_bl, c1_s2s_Wr,
           c2_s2r_Wl, c2_s2r_bl, c2_s2r_Wr, c2_r2r_Wl, c2_r2r_bl, c2_r2r_Wr,
           c2_s2s_Wl, c2_s2s_bl, c2_s2s_Wr,
           lin_W, lin_b):
    xp = jnp.pad(x_subject, ((0, N_PAD - NS), (0, 0)))
    npad = E_PAD - E
    pad_col = jnp.full((npad,), NS, jnp.int32)
    srcs = jnp.concatenate([ei_s2s[0], pad_col]).reshape(ROWS_W, K)
    dsts = jnp.concatenate([ei_s2s[1], pad_col]).reshape(ROWS_W, K)
    z72 = jnp.zeros((N_PAD, WC), _f32)
    z64 = jnp.zeros((N_PAD, H), _f32)
    ecol = jnp.pad(jnp.ones((1, 1), _f32), ((0, 0), (0, WC - H - 1)))
    bl1 = c1_s2s_bl.reshape(1, H)
    bl2 = c2_s2s_bl.reshape(1, H)
    wlin = jnp.pad(lin_W, ((0, 0), (0, 128 - O)))
    blin = jnp.pad(lin_b, ((0, 128 - O))).reshape(1, 128)

    y1, z1 = pl.pallas_call(
        _pre_body,
        out_shape=[jax.ShapeDtypeStruct((N_PAD, WC), _f32),
                   jax.ShapeDtypeStruct((N_PAD, H), _f32)],
    )(xp, c1_s2s_Wl, c1_s2s_Wr, ecol)

    s1p = _seg_sum_kernel(WC)(y1, srcs, dsts, z72)

    y2, z2, cntc = pl.pallas_call(
        _mid_body,
        out_shape=[jax.ShapeDtypeStruct((N_PAD, H), _f32),
                   jax.ShapeDtypeStruct((N_PAD, H), _f32),
                   jax.ShapeDtypeStruct((N_PAD, 8), _f32)],
    )(s1p, z1, bl1, c2_s2s_Wl, c2_s2s_Wr)

    s2p = _seg_sum_kernel(H)(y2, srcs, dsts, z64)

    outp = pl.pallas_call(
        _fin_body,
        out_shape=jax.ShapeDtypeStruct((N_PAD, 128), _f32),
    )(s2p, cntc, z2, bl2, wlin, blin)

    return outp[:NS, :O]
